# baseline (speedup 1.0000x reference)
import functools

import jax
import jax.numpy as jnp
from jax import lax
from jax.experimental import pallas as pl
from jax.experimental.pallas import tpu as pltpu
from jax.experimental.pallas import tpu_sc as plsc

VOCAB = 100000
DIM = 128
B = 4096
S = 200

NC = 2
NS = 16
NW = NC * NS

CHUNK = 100
TOK = B * S
NROW = TOK // CHUNK
ROWS_PER_W = NROW // NW
NBUF = 4


def _sc_body(ids_hbm, table_hbm, pos_hbm, out_hbm, idx_v, pos_v, bufs, gsems, osems):
    wid = lax.axis_index("s") * NC + lax.axis_index("c")
    row0 = wid * ROWS_PER_W

    pltpu.sync_copy(ids_hbm.at[pl.ds(row0, ROWS_PER_W)], idx_v)
    pltpu.sync_copy(pos_hbm, pos_v)

    def start_gather(c, b):
        pltpu.async_copy(table_hbm.at[idx_v.at[c]], bufs[b], gsems[b])

    def wait_gather(c, b):
        pltpu.make_async_copy(table_hbm.at[idx_v.at[c]], bufs[b], gsems[b]).wait()

    def start_out(c, b):
        pltpu.async_copy(bufs[b], out_hbm.at[row0 + c], osems[b])

    def wait_out(c, b):
        pltpu.make_async_copy(bufs[b], out_hbm.at[row0 + c], osems[b]).wait()

    for b in range(NBUF - 1):
        start_gather(b, b)

    def group_body(g, _):
        for b in range(NBUF):
            c = g * NBUF + b
            bn = (b + NBUF - 1) % NBUF

            wait_gather(c, b)
            half = lax.rem(c, 2)
            buf = bufs[b]

            def add_body(r, _):
                for j in range(DIM // 16):
                    sl = pl.ds(j * 16, 16)
                    plsc.addupdate(buf.at[r, sl], pos_v[half, r, sl])
                return ()

            lax.fori_loop(0, CHUNK, add_body, (), unroll=4)

            start_out(c, b)

            @pl.when(c >= 1)
            def _wait_prev_out():
                wait_out(c - 1, bn)

            @pl.when(c + NBUF - 1 < ROWS_PER_W)
            def _fire():
                start_gather(c + NBUF - 1, bn)
        return ()

    lax.fori_loop(0, ROWS_PER_W // NBUF, group_body, ())

    wait_out(ROWS_PER_W - 1, NBUF - 1)


@functools.partial(jax.jit, static_argnames=())
def kernel(token_ids, token_table, pos_table):
    ids = token_ids.astype(jnp.int32).reshape(NROW, CHUNK)
    pos = pos_table.astype(jnp.float32).reshape(S // CHUNK, CHUNK, DIM)

    mesh = plsc.VectorSubcoreMesh(
        core_axis_name="c", subcore_axis_name="s", num_cores=NC,
        num_subcores=NS)
    out = pl.kernel(
        _sc_body,
        out_type=jax.ShapeDtypeStruct((NROW, CHUNK, DIM), jnp.float32),
        mesh=mesh,
        scratch_types=[
            pltpu.VMEM((ROWS_PER_W, CHUNK), jnp.int32),
            pltpu.VMEM((S // CHUNK, CHUNK, DIM), jnp.float32),
            [pltpu.VMEM((CHUNK, DIM), jnp.float32) for _ in range(NBUF)],
            [pltpu.SemaphoreType.DMA for _ in range(NBUF)],
            [pltpu.SemaphoreType.DMA for _ in range(NBUF)],
        ],
    )(ids, token_table, pos)
    return out.reshape(B, S, DIM)


# speedup vs baseline: 1.7958x; 1.7958x over previous
import functools

import jax
import jax.numpy as jnp
from jax import lax
from jax.experimental import pallas as pl
from jax.experimental.pallas import tpu as pltpu
from jax.experimental.pallas import tpu_sc as plsc

VOCAB = 100000
DIM = 128
B = 4096
S = 200

NC = 2
NS = 16
NW = NC * NS

HALF = S // 2
SEQ_PER_W = B // NW
NBUF = 2


def _sc_body(ids_hbm, table_hbm, pos_hbm, out_hbm, idx_v, pos_v, bufs, gsems, osems):
    wid = lax.axis_index("s") * NC + lax.axis_index("c")
    seq0 = wid * SEQ_PER_W

    pltpu.sync_copy(ids_hbm.at[pl.ds(wid * (2 * SEQ_PER_W), 2 * SEQ_PER_W)], idx_v)
    pltpu.sync_copy(pos_hbm, pos_v)

    def start_gather(q, b):
        pltpu.async_copy(table_hbm.at[idx_v.at[2 * q]],
                         bufs[b].at[pl.ds(0, HALF)], gsems[b])
        pltpu.async_copy(table_hbm.at[idx_v.at[2 * q + 1]],
                         bufs[b].at[pl.ds(HALF, HALF)], gsems[b])

    def wait_gather(q, b):
        pltpu.make_async_copy(table_hbm.at[idx_v.at[2 * q]],
                              bufs[b].at[pl.ds(0, HALF)], gsems[b]).wait()
        pltpu.make_async_copy(table_hbm.at[idx_v.at[2 * q + 1]],
                              bufs[b].at[pl.ds(HALF, HALF)], gsems[b]).wait()

    def start_out(q, b):
        pltpu.async_copy(bufs[b], out_hbm.at[seq0 + q], osems[b])

    def wait_out(q, b):
        pltpu.make_async_copy(bufs[b], out_hbm.at[seq0 + q], osems[b]).wait()

    def add_pos(b):
        buf = bufs[b]

        def add_body(r, _):
            for j in range(DIM // 16):
                sl = pl.ds(j * 16, 16)
                plsc.addupdate(buf.at[r, sl], pos_v[r, sl])
            return ()

        lax.fori_loop(0, S, add_body, (), unroll=4)

    start_gather(0, 0)

    def group_body(g, _):
        for b in range(NBUF):
            q = g * NBUF + b
            bn = 1 - b

            @pl.when(q >= 1)
            def _wait_prev_out():
                wait_out(q - 1, bn)

            @pl.when(q + 1 < SEQ_PER_W)
            def _fire():
                start_gather(q + 1, bn)

            wait_gather(q, b)
            add_pos(b)
            start_out(q, b)
        return ()

    lax.fori_loop(0, SEQ_PER_W // NBUF, group_body, ())
    wait_out(SEQ_PER_W - 1, 1)


@functools.partial(jax.jit, static_argnames=())
def kernel(token_ids, token_table, pos_table):
    ids = token_ids.astype(jnp.int32).reshape(2 * B, HALF)

    mesh = plsc.VectorSubcoreMesh(
        core_axis_name="c", subcore_axis_name="s", num_cores=NC,
        num_subcores=NS)
    return pl.kernel(
        _sc_body,
        out_type=jax.ShapeDtypeStruct((B, S, DIM), jnp.float32),
        mesh=mesh,
        scratch_types=[
            pltpu.VMEM((2 * SEQ_PER_W, HALF), jnp.int32),
            pltpu.VMEM((S, DIM), jnp.float32),
            [pltpu.VMEM((S, DIM), jnp.float32) for _ in range(NBUF)],
            [pltpu.SemaphoreType.DMA for _ in range(NBUF)],
            [pltpu.SemaphoreType.DMA for _ in range(NBUF)],
        ],
    )(ids, token_table, pos_table)
